# initial kernel scaffold (unmeasured)
import jax
import jax.numpy as jnp
from jax import lax
from jax.experimental import pallas as pl
from jax.experimental.pallas import tpu as pltpu


def kernel(
    x,
):
    def body(*refs):
        pass

    out_shape = jax.ShapeDtypeStruct(..., jnp.float32)
    return pl.pallas_call(body, out_shape=out_shape)(...)


# baseline (device time: 212408 ns/iter reference)
import jax
import jax.numpy as jnp
from jax import lax
from jax.experimental import pallas as pl
from jax.experimental.pallas import tpu as pltpu

P = 32


def kernel(x):
    M, N = x.shape
    C = M // P

    def body(x_ref, out_ref, comm_ref, sbuf_ref, rs_send, rs_recv, ag_send, ag_recv):
        me = lax.axis_index("i")
        left = jnp.mod(me - 1 + P, P)
        right = jnp.mod(me + 1, P)

        barrier_sem = pltpu.get_barrier_semaphore()
        for nbr in (left, right):
            pl.semaphore_signal(
                barrier_sem, inc=1,
                device_id=(nbr,), device_id_type=pl.DeviceIdType.MESH,
            )
        pl.semaphore_wait(barrier_sem, 2)

        for s in range(P - 1):
            off = jnp.mod(me - s, P) * C
            xc = x_ref[pl.ds(off, C), :]
            if s == 0:
                sbuf_ref[0] = xc.astype(jnp.bfloat16)
            else:
                sbuf_ref[s] = (
                    comm_ref[s - 1].astype(jnp.float32) + xc
                ).astype(jnp.bfloat16)
            rdma = pltpu.make_async_remote_copy(
                src_ref=sbuf_ref.at[s],
                dst_ref=comm_ref.at[s],
                send_sem=rs_send.at[s],
                recv_sem=rs_recv.at[s],
                device_id=(right,),
                device_id_type=pl.DeviceIdType.MESH,
            )
            rdma.start()
            rdma.wait()

        own = jnp.mod(me + 1, P) * C
        out_ref[pl.ds(own, C), :] = (
            comm_ref[P - 2].astype(jnp.float32) + x_ref[pl.ds(own, C), :]
        ).astype(jnp.bfloat16)

        for t in range(P - 1):
            coff = jnp.mod(me + 1 - t, P) * C
            rdma = pltpu.make_async_remote_copy(
                src_ref=out_ref.at[pl.ds(coff, C)],
                dst_ref=out_ref.at[pl.ds(coff, C)],
                send_sem=ag_send.at[t],
                recv_sem=ag_recv.at[t],
                device_id=(right,),
                device_id_type=pl.DeviceIdType.MESH,
            )
            rdma.start()
            rdma.wait()

    return pl.pallas_call(
        body,
        out_shape=jax.ShapeDtypeStruct((M, N), jnp.bfloat16),
        in_specs=[pl.BlockSpec(memory_space=pltpu.VMEM)],
        out_specs=pl.BlockSpec(memory_space=pltpu.VMEM),
        scratch_shapes=[
            pltpu.VMEM((P - 1, C, N), jnp.bfloat16),
            pltpu.VMEM((P - 1, C, N), jnp.bfloat16),
            pltpu.SemaphoreType.DMA((P - 1,)),
            pltpu.SemaphoreType.DMA((P - 1,)),
            pltpu.SemaphoreType.DMA((P - 1,)),
            pltpu.SemaphoreType.DMA((P - 1,)),
        ],
        compiler_params=pltpu.CompilerParams(collective_id=0),
    )(x)


# device time: 122802 ns/iter; 1.7297x vs baseline; 1.7297x over previous
import jax
import jax.numpy as jnp
from jax import lax
from jax.experimental import pallas as pl
from jax.experimental.pallas import tpu as pltpu

P = 32
XOR_RS = (1, 8, 2, 4, 16)


def kernel(x):
    M, N = x.shape
    sizes = [M >> (k + 1) for k in range(5)]
    slot_off = [sum(sizes[:k]) for k in range(5)]
    comm_rows = sum(sizes)

    def body(x_ref, out_ref, comm_ref, rs_send, rs_recv, ag_send, ag_recv):
        me = lax.axis_index("i")

        barrier_sem = pltpu.get_barrier_semaphore()
        for v in XOR_RS:
            pl.semaphore_signal(
                barrier_sem, inc=1,
                device_id=(jnp.bitwise_xor(me, v),),
                device_id_type=pl.DeviceIdType.MESH,
            )
        pl.semaphore_wait(barrier_sem, 5)

        out_ref[...] = x_ref[...].astype(jnp.bfloat16)

        off = jnp.int32(0)
        for k, v in enumerate(XOR_RS):
            sz = sizes[k]
            partner = jnp.bitwise_xor(me, v)
            mybit = (jnp.bitwise_and(me, v) != 0).astype(jnp.int32)
            send_off = off + (1 - mybit) * sz
            keep_off = off + mybit * sz
            rdma = pltpu.make_async_remote_copy(
                src_ref=out_ref.at[pl.ds(send_off, sz)],
                dst_ref=comm_ref.at[pl.ds(slot_off[k], sz)],
                send_sem=rs_send.at[k],
                recv_sem=rs_recv.at[k],
                device_id=(partner,),
                device_id_type=pl.DeviceIdType.MESH,
            )
            rdma.start()
            rdma.wait()
            out_ref[pl.ds(keep_off, sz)] = (
                out_ref[pl.ds(keep_off, sz)].astype(jnp.float32)
                + comm_ref[pl.ds(slot_off[k], sz)].astype(jnp.float32)
            ).astype(jnp.bfloat16)
            off = keep_off


        for j in range(5):
            v = XOR_RS[4 - j]
            cur = sizes[4] << j
            partner = jnp.bitwise_xor(me, v)
            mybit = (jnp.bitwise_and(me, v) != 0).astype(jnp.int32)
            rdma = pltpu.make_async_remote_copy(
                src_ref=out_ref.at[pl.ds(off, cur)],
                dst_ref=out_ref.at[pl.ds(off, cur)],
                send_sem=ag_send.at[j],
                recv_sem=ag_recv.at[j],
                device_id=(partner,),
                device_id_type=pl.DeviceIdType.MESH,
            )
            rdma.start()
            rdma.wait()
            off = off - mybit * cur

    return pl.pallas_call(
        body,
        out_shape=jax.ShapeDtypeStruct((M, N), jnp.bfloat16),
        in_specs=[pl.BlockSpec(memory_space=pltpu.VMEM)],
        out_specs=pl.BlockSpec(memory_space=pltpu.VMEM),
        scratch_shapes=[
            pltpu.VMEM((comm_rows, N), jnp.bfloat16),
            pltpu.SemaphoreType.DMA((5,)),
            pltpu.SemaphoreType.DMA((5,)),
            pltpu.SemaphoreType.DMA((5,)),
            pltpu.SemaphoreType.DMA((5,)),
        ],
        compiler_params=pltpu.CompilerParams(collective_id=0),
    )(x)


# device time: 84208 ns/iter; 2.5224x vs baseline; 1.4583x over previous
import jax
import jax.numpy as jnp
from jax import lax
from jax.experimental import pallas as pl
from jax.experimental.pallas import tpu as pltpu

P = 32
MASKS = (1, 2, 4, 8, 16)
ORDERS = ((1, 8, 2, 4, 16), (8, 1, 16, 2, 4))


def kernel(x):
    M, N = x.shape
    R = M // 2
    sizes = [R >> (k + 1) for k in range(5)]
    stream_rows = sum(sizes)

    def slot(s, k):
        return s * stream_rows + sum(sizes[:k])

    def body(x_ref, out_ref, comm_ref, rs_send, rs_recv, ag_send, ag_recv):
        me = lax.axis_index("i")

        barrier_sem = pltpu.get_barrier_semaphore()
        for v in MASKS:
            pl.semaphore_signal(
                barrier_sem, inc=1,
                device_id=(jnp.bitwise_xor(me, v),),
                device_id_type=pl.DeviceIdType.MESH,
            )
        pl.semaphore_wait(barrier_sem, 5)

        out_ref[...] = x_ref[...].astype(jnp.bfloat16)

        off = [jnp.int32(0), jnp.int32(R)]
        for k in range(5):
            sz = sizes[k]
            rdmas = []
            keep = []
            for s in (0, 1):
                v = ORDERS[s][k]
                partner = jnp.bitwise_xor(me, v)
                mybit = (jnp.bitwise_and(me, v) != 0).astype(jnp.int32)
                send_off = off[s] + (1 - mybit) * sz
                keep.append(off[s] + mybit * sz)
                rdma = pltpu.make_async_remote_copy(
                    src_ref=out_ref.at[pl.ds(send_off, sz)],
                    dst_ref=comm_ref.at[pl.ds(slot(s, k), sz)],
                    send_sem=rs_send.at[s * 5 + k],
                    recv_sem=rs_recv.at[s * 5 + k],
                    device_id=(partner,),
                    device_id_type=pl.DeviceIdType.MESH,
                )
                rdma.start()
                rdmas.append(rdma)
            for s in (0, 1):
                rdmas[s].wait()
                out_ref[pl.ds(keep[s], sz)] = (
                    out_ref[pl.ds(keep[s], sz)].astype(jnp.float32)
                    + comm_ref[pl.ds(slot(s, k), sz)].astype(jnp.float32)
                ).astype(jnp.bfloat16)
                off[s] = keep[s]


        for j in range(5):
            cur = sizes[4] << j
            rdmas = []
            bits = []
            for s in (0, 1):
                v = ORDERS[s][4 - j]
                partner = jnp.bitwise_xor(me, v)
                bits.append((jnp.bitwise_and(me, v) != 0).astype(jnp.int32))
                rdma = pltpu.make_async_remote_copy(
                    src_ref=out_ref.at[pl.ds(off[s], cur)],
                    dst_ref=out_ref.at[pl.ds(off[s], cur)],
                    send_sem=ag_send.at[s * 5 + j],
                    recv_sem=ag_recv.at[s * 5 + j],
                    device_id=(partner,),
                    device_id_type=pl.DeviceIdType.MESH,
                )
                rdma.start()
                rdmas.append(rdma)
            for s in (0, 1):
                rdmas[s].wait()
                off[s] = off[s] - bits[s] * cur

    return pl.pallas_call(
        body,
        out_shape=jax.ShapeDtypeStruct((M, N), jnp.bfloat16),
        in_specs=[pl.BlockSpec(memory_space=pltpu.VMEM)],
        out_specs=pl.BlockSpec(memory_space=pltpu.VMEM),
        scratch_shapes=[
            pltpu.VMEM((2 * stream_rows, N), jnp.bfloat16),
            pltpu.SemaphoreType.DMA((10,)),
            pltpu.SemaphoreType.DMA((10,)),
            pltpu.SemaphoreType.DMA((10,)),
            pltpu.SemaphoreType.DMA((10,)),
        ],
        compiler_params=pltpu.CompilerParams(collective_id=0),
    )(x)
